# baseline (speedup 1.0000x reference)
import functools

import jax
import jax.numpy as jnp
from jax import lax
from jax.experimental import pallas as pl
from jax.experimental.pallas import tpu as pltpu
from jax.experimental.pallas import tpu_sc as plsc

C = 128
TF = 512
NW = 32
CB = 64
NV = C // 16


def _pad_to(f):
    m = NW * CB
    return ((f + m - 1) // m) * m


def _tc1_body(xu, wu, bup, ys, yt):
    y = jnp.dot(xu[...].astype(jnp.bfloat16), wu[...],
                preferred_element_type=jnp.float32)
    y = y + bup[...]
    ys[...] = y[:, 0:C]
    yt[0] = y[:, C:2 * C]
    yt[1] = y[:, 2 * C:3 * C]
    yt[2] = y[:, 3 * C:4 * C]


def _tc2_body(x1, xd, wa, wb, bcp, zs, zt):
    z = (jnp.dot(x1[...].astype(jnp.bfloat16), wa[...],
                 preferred_element_type=jnp.float32)
         + jnp.dot(xd[...].astype(jnp.bfloat16), wb[...],
                   preferred_element_type=jnp.float32))
    z = z + bcp[...]
    zs[...] = z[:, 0:C]
    zt[0] = z[:, C:2 * C]
    zt[1] = z[:, 2 * C:3 * C]
    zt[2] = z[:, 3 * C:4 * C]


def _tc3_body(x2, w2, b2p, vs, vt):
    v = jnp.dot(x2[...].astype(jnp.bfloat16), w2[...],
                preferred_element_type=jnp.float32)
    v = v + b2p[...]
    vs[...] = v[:, 0:C]
    vt[0] = v[:, C:2 * C]
    vt[1] = v[:, 2 * C:3 * C]
    vt[2] = v[:, 3 * C:4 * C]


def _row_spec(bf):
    return pl.BlockSpec((bf, C), lambda i: (i, 0))


def _tap_spec():
    return pl.BlockSpec((3, TF, C), lambda i: (0, i, 0))


def _full_spec(shape):
    return pl.BlockSpec(shape, lambda i: (0,) * len(shape))


def _sc_acc_body(n_chunks, relu, tap, self_t, idxh, out,
                 idx_v, bt, bs, bo, gsems, ssems, wsems):
    rows_per_w = n_chunks * CB
    wid = lax.axis_index("s") * 2 + lax.axis_index("c")
    pltpu.sync_copy(idxh.at[wid], idx_v)

    def issue(ci, s):
        ds = [pltpu.async_copy(tap.at[idx_v.at[ci, k]], bt[s].at[k], gsems[s])
              for k in range(3)]
        base = wid * rows_per_w + ci * CB
        ds.append(pltpu.async_copy(self_t.at[pl.ds(base, CB)], bs[s], ssems[s]))
        return ds

    def accumulate(s):
        bt_s, bs_s, bo_s = bt[s], bs[s], bo[s]

        def body(r, carry):
            for u in range(NV):
                sl = pl.ds(u * 16, 16)
                acc = bs_s[r, sl] + bt_s[0, r, sl]
                acc = acc + bt_s[1, r, sl]
                acc = acc + bt_s[2, r, sl]
                if relu:
                    acc = jnp.maximum(acc, 0.0)
                bo_s[r, sl] = acc
            return carry

        lax.fori_loop(0, CB, body, 0)

    pend = [None, None]
    wr = [None, None]
    pend[0] = issue(0, 0)
    for ci in range(n_chunks):
        s = ci % 2
        if ci + 1 < n_chunks:
            pend[1 - s] = issue(ci + 1, 1 - s)
        for d in pend[s]:
            d.wait()
        if wr[s] is not None:
            wr[s].wait()
        accumulate(s)
        base = wid * rows_per_w + ci * CB
        wr[s] = pltpu.async_copy(bo[s], out.at[pl.ds(base, CB)], wsems[s])
    for s in (0, 1):
        if wr[s] is not None:
            wr[s].wait()


def _sc_acc(tap3, self_t, idxh, relu):
    n_chunks = idxh.shape[1]
    fp = self_t.shape[0]
    mesh = plsc.VectorSubcoreMesh(core_axis_name="c", subcore_axis_name="s")
    return pl.kernel(
        functools.partial(_sc_acc_body, n_chunks, relu),
        out_type=jax.ShapeDtypeStruct((fp, C), jnp.float32),
        mesh=mesh,
        scratch_types=[
            pltpu.VMEM((n_chunks, 3, CB), jnp.int32),
            [pltpu.VMEM((3, CB, C), jnp.float32) for _ in range(2)],
            [pltpu.VMEM((CB, C), jnp.float32) for _ in range(2)],
            [pltpu.VMEM((CB, C), jnp.float32) for _ in range(2)],
            [pltpu.SemaphoreType.DMA for _ in range(2)],
            [pltpu.SemaphoreType.DMA for _ in range(2)],
            [pltpu.SemaphoreType.DMA for _ in range(2)],
        ],
    )(tap3, self_t, idxh)


def kernel(from_up, from_down, gemm_faces, W_up, b_up, W_c1, b_c1, W_c2, b_c2):
    f = from_up.shape[2]
    fp = _pad_to(f)
    grid = fp // TF
    n_chunks = fp // (NW * CB)

    xu = from_up[0].T
    xd = from_down[0].T

    wu = W_up[:, :, 0, :].transpose(1, 2, 0).reshape(C, 4 * C)
    wa = W_c1[:, :C, 0, :].transpose(1, 2, 0).reshape(C, 4 * C)
    wb = W_c1[:, C:, 0, :].transpose(1, 2, 0).reshape(C, 4 * C)
    w2 = W_c2[:, :, 0, :].transpose(1, 2, 0).reshape(C, 4 * C)
    w2 = w2.at[:, :C].add(jnp.eye(C, dtype=jnp.float32))
    wu = wu.astype(jnp.bfloat16)
    wa = wa.astype(jnp.bfloat16)
    wb = wb.astype(jnp.bfloat16)
    w2 = w2.astype(jnp.bfloat16)
    zpad = jnp.zeros((3 * C,), jnp.float32)
    bup = jnp.concatenate([b_up, zpad]).reshape(1, 4 * C)
    bcp = jnp.concatenate([b_c1, zpad]).reshape(1, 4 * C)
    b2p = jnp.concatenate([b_c2, zpad]).reshape(1, 4 * C)

    nbr = jnp.pad(gemm_faces[0], ((0, fp - f), (0, 0))).T
    offs = (jnp.arange(3, dtype=jnp.int32) * fp)[:, None]
    idx = (nbr + offs).reshape(3, NW, n_chunks, CB).transpose(1, 2, 0, 3)

    row = jax.ShapeDtypeStruct((fp, C), jnp.float32)
    tap = jax.ShapeDtypeStruct((3, fp, C), jnp.float32)

    ys, yt = pl.pallas_call(
        _tc1_body,
        grid=(grid,),
        in_specs=[_row_spec(TF), _full_spec((C, 4 * C)),
                  _full_spec((1, 4 * C))],
        out_specs=[_row_spec(TF), _tap_spec()],
        out_shape=[row, tap],
    )(xu, wu, bup)
    x1 = _sc_acc(yt.reshape(3 * fp, C), ys, idx, relu=False)

    zs, zt = pl.pallas_call(
        _tc2_body,
        grid=(grid,),
        in_specs=[_row_spec(TF), _row_spec(TF),
                  _full_spec((C, 4 * C)), _full_spec((C, 4 * C)),
                  _full_spec((1, 4 * C))],
        out_specs=[_row_spec(TF), _tap_spec()],
        out_shape=[row, tap],
    )(x1, xd, wa, wb, bcp)
    x2 = _sc_acc(zt.reshape(3 * fp, C), zs, idx, relu=True)

    vs, vt = pl.pallas_call(
        _tc3_body,
        grid=(grid,),
        in_specs=[_row_spec(TF), _full_spec((C, 4 * C)),
                  _full_spec((1, 4 * C))],
        out_specs=[_row_spec(TF), _tap_spec()],
        out_shape=[row, tap],
    )(x2, w2, b2p)
    out = _sc_acc(vt.reshape(3 * fp, C), vs, idx, relu=True)

    return out[:f].T[None]
